# Initial kernel scaffold; baseline (speedup 1.0000x reference)
import jax
import jax.numpy as jnp
from jax.experimental import pallas as pl


def kernel(coords, nogeo_khot, nogeo_ids, W_spa, b_spa, nogeo_table):
    raise NotImplementedError("write your pallas kernel here")


# speedup vs baseline: 1.3408x; 1.3408x over previous
import functools

import jax
import jax.numpy as jnp
from jax import lax
from jax.experimental import pallas as pl
from jax.experimental.pallas import tpu as pltpu
from jax.experimental.pallas import tpu_sc as plsc

B = 16384
D = 64
CH = 128


@functools.cache
def _make_sc_gather():
    info = plsc.get_sparse_core_info()
    nw = info.num_cores * info.num_subcores
    b_per_w = B // nw
    n_ch = b_per_w // CH
    mesh = plsc.VectorSubcoreMesh(core_axis_name="c", subcore_axis_name="s")

    @functools.partial(
        pl.kernel,
        mesh=mesh,
        out_type=jax.ShapeDtypeStruct((B, D), jnp.float32),
        compiler_params=pltpu.CompilerParams(use_tc_tiling_on_sc=False),
        scratch_types=[
            pltpu.VMEM((n_ch, CH), jnp.int32),
            pltpu.VMEM((b_per_w, D), jnp.float32),
            pltpu.SemaphoreType.DMA,
        ],
    )
    def gather_kernel(idx_hbm, table_hbm, out_hbm, idx_v, rows_v, sem):
        wid = lax.axis_index("s") * info.num_cores + lax.axis_index("c")
        base = wid * b_per_w
        pltpu.sync_copy(idx_hbm.at[pl.ds(wid * n_ch, n_ch)], idx_v)
        copies = [
            pltpu.async_copy(
                table_hbm.at[idx_v.at[j]], rows_v.at[pl.ds(j * CH, CH)], sem
            )
            for j in range(n_ch)
        ]
        for c in copies:
            c.wait()
        pltpu.sync_copy(rows_v, out_hbm.at[pl.ds(base, b_per_w)])

    return gather_kernel


def _tc_body(g_ref, c_ref, m_ref, w_ref, b_ref, o_ref):
    g = g_ref[...]
    c = c_ref[...]
    m = m_ref[...]
    w = w_ref[...]
    bb = b_ref[...]
    spa = c[:, 0:1] * w[0:1, :] + c[:, 1:2] * w[1:2, :] + bb
    gn = g * lax.rsqrt(jnp.sum(g * g, axis=1, keepdims=True))
    v = spa * (1.0 - m) + gn * m
    v = v * lax.rsqrt(jnp.sum(v * v, axis=1, keepdims=True))
    o_ref[...] = v.T


def kernel(coords, nogeo_khot, nogeo_ids, W_spa, b_spa, nogeo_table):
    idx2 = nogeo_ids.reshape(B // CH, CH)
    gathered = _make_sc_gather()(idx2, nogeo_table)

    coords2 = coords.reshape(B, 2)
    maskf = nogeo_khot.astype(jnp.float32).reshape(B, 1)
    b2 = b_spa.reshape(1, D)

    bb = 1024
    return pl.pallas_call(
        _tc_body,
        grid=(B // bb,),
        in_specs=[
            pl.BlockSpec((bb, D), lambda i: (i, 0)),
            pl.BlockSpec((bb, 2), lambda i: (i, 0)),
            pl.BlockSpec((bb, 1), lambda i: (i, 0)),
            pl.BlockSpec((2, D), lambda i: (0, 0)),
            pl.BlockSpec((1, D), lambda i: (0, 0)),
        ],
        out_specs=pl.BlockSpec((D, bb), lambda i: (0, i)),
        out_shape=jax.ShapeDtypeStruct((D, B), jnp.float32),
    )(gathered, coords2, maskf, W_spa, b2)
